# baseline (speedup 1.0000x reference)
import functools

import jax
import jax.numpy as jnp
from jax import lax
from jax.experimental import pallas as pl
from jax.experimental.pallas import tpu as pltpu
from jax.experimental.pallas import tpu_sc as plsc

_D = 128
_NC = 2
_NS = 16
_NW = _NC * _NS
_NBUF = 4
_L = 16


@functools.partial(jax.jit, static_argnames=("bsz", "lsz"))
def _gather(idx, table, *, bsz, lsz):
    rows_per_w = bsz // _NW
    per_w = rows_per_w * lsz
    splits = []
    off = 0
    while off < lsz:
        ck = min(128, lsz - off)
        splits.append((off, ck))
        off += ck
    cols = list(range(0, lsz - _L + 1, _L))
    if cols[-1] != lsz - _L:
        cols.append(lsz - _L)
    mesh = plsc.VectorSubcoreMesh(core_axis_name="c", subcore_axis_name="s")

    @functools.partial(
        pl.kernel,
        out_type=jax.ShapeDtypeStruct((bsz, lsz, _D), jnp.float32),
        mesh=mesh,
        scratch_types=[
            pltpu.VMEM((rows_per_w, lsz), jnp.int32),
            pltpu.VMEM((per_w,), jnp.int32),
            pltpu.VMEM((_NBUF, lsz, _D), jnp.float32),
            pltpu.SemaphoreType.DMA((_NBUF,)),
            pltpu.SemaphoreType.DMA((_NBUF,)),
        ],
    )
    def body(idx_hbm, table_hbm, out_hbm, idx2_v, idx_v, rows_v, gsem, ssem):
        wid = lax.axis_index("s") * _NC + lax.axis_index("c")
        rbase = pl.multiple_of(wid * rows_per_w, 8)
        pltpu.sync_copy(idx_hbm.at[pl.ds(rbase, rows_per_w)], idx2_v)

        for r in range(rows_per_w):
            for c in cols:
                idx_v[pl.ds(r * lsz + c, _L)] = idx2_v[r, pl.ds(c, _L)]

        def start_gathers(r, b):
            for off, ck in splits:
                pltpu.async_copy(
                    table_hbm.at[idx_v.at[pl.ds(
                        pl.multiple_of(r * lsz + off, 8), ck)]],
                    rows_v.at[b, pl.ds(off, ck)], gsem.at[b])

        def start_store(r, b):
            pltpu.make_async_copy(out_hbm.at[rbase + r], rows_v.at[b],
                                  gsem.at[b]).wait()
            pltpu.async_copy(rows_v.at[b], out_hbm.at[rbase + r], ssem.at[b])

        def wait_store(r, b):
            pltpu.make_async_copy(rows_v.at[b], out_hbm.at[rbase + r],
                                  ssem.at[b]).wait()

        @pl.loop(0, rows_per_w)
        def _(r):
            b = lax.rem(r, _NBUF)

            @pl.when(r >= _NBUF)
            def _():
                wait_store(r - _NBUF, b)

            start_gathers(r, b)

            @pl.when(r >= 1)
            def _():
                start_store(r - 1, lax.rem(r - 1, _NBUF))

        start_store(rows_per_w - 1, (rows_per_w - 1) % _NBUF)
        for r in range(rows_per_w - _NBUF, rows_per_w):
            wait_store(r, r % _NBUF)

    return body(idx, table)


def kernel(index, table):
    b, l = index.shape
    return _gather(index, table, bsz=b, lsz=l)


# speedup vs baseline: 1.0125x; 1.0125x over previous
import functools

import jax
import jax.numpy as jnp
from jax import lax
from jax.experimental import pallas as pl
from jax.experimental.pallas import tpu as pltpu
from jax.experimental.pallas import tpu_sc as plsc

_D = 128
_NC = 2
_NS = 16
_NW = _NC * _NS
_CHUNK = 64
_NBUF = 10
_LAG = 5


@functools.partial(jax.jit, static_argnames=("n",))
def _gather(idx3d, table, *, n):
    per_w = n // _NW
    nch = per_w // _CHUNK
    ngroups = nch // _NBUF
    mesh = plsc.VectorSubcoreMesh(core_axis_name="c", subcore_axis_name="s")

    @functools.partial(
        pl.kernel,
        out_type=jax.ShapeDtypeStruct((n, _D), jnp.float32),
        mesh=mesh,
        scratch_types=[
            pltpu.VMEM((nch, _CHUNK), jnp.int32),
            pltpu.VMEM((_NBUF, _CHUNK, _D), jnp.float32),
            pltpu.SemaphoreType.DMA((_NBUF,)),
            pltpu.SemaphoreType.DMA((_NBUF,)),
        ],
    )
    def body(idx_hbm, table_hbm, out_hbm, idx_v, rows_v, gsem, ssem):
        wid = lax.axis_index("s") * _NC + lax.axis_index("c")
        base = wid * per_w
        pltpu.sync_copy(idx_hbm.at[wid], idx_v)

        def start_gather(j, b):
            pltpu.async_copy(table_hbm.at[idx_v.at[j]], rows_v.at[b],
                             gsem.at[b])

        def wait_store(i, b):
            pltpu.make_async_copy(
                rows_v.at[b],
                out_hbm.at[pl.ds(base + i * _CHUNK, _CHUNK)],
                ssem.at[b]).wait()

        def start_store(i, b):
            pltpu.make_async_copy(table_hbm.at[idx_v.at[i]], rows_v.at[b],
                                  gsem.at[b]).wait()
            pltpu.async_copy(rows_v.at[b],
                             out_hbm.at[pl.ds(base + i * _CHUNK, _CHUNK)],
                             ssem.at[b])

        def do_group(g, first):
            for b in range(_NBUF):
                j = g * _NBUF + b
                if not first:
                    wait_store(j - _NBUF, b)
                start_gather(j, b)
                if not (first and b < _LAG):
                    start_store(j - _LAG, (b - _LAG) % _NBUF)

        do_group(0, True)

        @pl.loop(1, ngroups)
        def _(g):
            do_group(g, False)

        for i in range(nch - _LAG, nch):
            start_store(i, i % _NBUF)
        for b in range(_NBUF):
            wait_store(nch - _NBUF + b, b)

    return body(idx3d, table)


def kernel(index, table):
    b, l = index.shape
    n = b * l
    idx3d = index.reshape(_NW, n // (_NW * _CHUNK), _CHUNK)
    out = _gather(idx3d, table, n=n)
    return out.reshape(b, l, _D)
